# baseline (speedup 1.0000x reference)
import math

import jax
import jax.numpy as jnp
from jax import lax
from jax.experimental import pallas as pl
from jax.experimental.pallas import tpu as pltpu
from jax.experimental.pallas import tpu_sc as plsc

DEPTH = 11
N_NODES = 2 ** (DEPTH + 1) - 1
WIDTH = 2048
OUT_WIDTH = 2048
N_LEVELS = DEPTH + 1

L = 16
NW = 32
CHUNK = 16
HALF = 8
KCH = WIDTH // L
T_TOTAL = 8192
TPW = T_TOTAL // NW
UNROLL = 8
OGRP = 4


def _gelu_exact(z):
    u = z * jnp.float32(1.0 / math.sqrt(2.0))
    au = jnp.abs(u)
    t = jnp.float32(1.0) / (jnp.float32(1.0) + jnp.float32(0.3275911) * au)
    poly = jnp.float32(1.061405429)
    poly = poly * t + jnp.float32(-1.453152027)
    poly = poly * t + jnp.float32(1.421413741)
    poly = poly * t + jnp.float32(-0.284496736)
    poly = poly * t + jnp.float32(0.254829592)
    poly = poly * t
    erf_abs = jnp.float32(1.0) - poly * jnp.exp(-(au * au))
    erf = jnp.where(u >= 0, erf_abs, -erf_abs)
    return z * jnp.float32(0.5) * (jnp.float32(1.0) + erf)


_DNUMS = lax.GatherDimensionNumbers(
    offset_dims=(), collapsed_slice_dims=(0,), start_index_map=(0,))


def _lane_sum_splat(v):
    lanes = jnp.arange(L, dtype=jnp.int32)
    for sh in (8, 4, 2, 1):
        idx = (lanes ^ sh)[:, None]
        v = v + lax.gather(v, idx, _DNUMS, (1,),
                           mode=lax.GatherScatterMode.PROMISE_IN_BOUNDS)
    return v


def _level_body(x_hbm, win_hbm, bvals_hbm, idx_hbm, act_hbm, lg_hbm,
                x_v0, x_v1, w_v0, w_v1, idx_v, bv_v, st_v, lst_v,
                semw0, semw1, semx0, semx1):
    info = plsc.get_sparse_core_info()
    wid = lax.axis_index("s") * info.num_cores + lax.axis_index("c")
    tok0 = wid * TPW
    lanes = jnp.arange(L, dtype=jnp.int32)
    NH = TPW // HALF
    wbufs = (w_v0, w_v1)
    xbufs = (x_v0, x_v1)
    wsems = (semw0, semw1)
    xsems = (semx0, semx1)

    pltpu.sync_copy(idx_hbm.at[pl.ds(tok0, TPW)], idx_v)
    pltpu.sync_copy(bvals_hbm.at[pl.ds(tok0, TPW)], bv_v)

    def issue(g, par):
        idx_g = idx_v.at[pl.ds(g * HALF, HALF)]
        pltpu.async_copy(win_hbm.at[idx_g], wbufs[par], wsems[par])
        pltpu.async_copy(x_hbm.at[pl.ds(tok0 + g * HALF, HALF)],
                         xbufs[par], xsems[par])

    issue(0, 0)

    def chunk_body(c, _):
        logits = jnp.zeros((L,), jnp.float32)
        for h in (0, 1):
            g = 2 * c + h
            w_v, x_v = wbufs[h], xbufs[h]
            gn = jnp.minimum(g + 1, NH - 1)
            issue(gn, 1 - h)
            idx_g = idx_v.at[pl.ds(gn * HALF, HALF)]
            pltpu.make_async_copy(win_hbm.at[idx_g], w_v, wsems[h]).wait()
            pltpu.make_async_copy(
                x_hbm.at[pl.ds(tok0, HALF)], x_v, xsems[h]).wait()

            for t in range(HALF):
                def kb(k, acc, t=t, w_v=w_v, x_v=x_v):
                    for u in range(UNROLL):
                        s = pl.ds((k * UNROLL + u) * L, L)
                        acc = acc + x_v[t, s] * w_v[t, s]
                    return acc
                acc = lax.fori_loop(0, KCH // UNROLL, kb,
                                    jnp.zeros((L,), jnp.float32))
                logits = jnp.where(lanes == h * HALF + t,
                                   _lane_sum_splat(acc), logits)

        lg = logits + bv_v[pl.ds(c * CHUNK, CHUNK)]
        act = _gelu_exact(lg)
        st_v[pl.ds(c * CHUNK, CHUNK)] = act
        lst_v[pl.ds(c * CHUNK, CHUNK)] = lg
        return 0

    lax.fori_loop(0, TPW // CHUNK, chunk_body, 0)
    idx0 = idx_v.at[pl.ds(0, HALF)]
    pltpu.make_async_copy(win_hbm.at[idx0], w_v0, semw0).wait()
    pltpu.make_async_copy(x_hbm.at[pl.ds(tok0, HALF)], x_v0, semx0).wait()
    pltpu.sync_copy(st_v, act_hbm.at[pl.ds(tok0, TPW)])
    pltpu.sync_copy(lst_v, lg_hbm.at[pl.ds(tok0, TPW)])


def _out_body(wout_hbm, idxf_hbm, acts_hbm, out_hbm,
              g_v0, g_v1, out_v, idx_v, a_v, semg0, semg1):
    info = plsc.get_sparse_core_info()
    wid = lax.axis_index("s") * info.num_cores + lax.axis_index("c")
    tok0 = wid * TPW
    gbufs = (g_v0, g_v1)
    gsems = (semg0, semg1)

    pltpu.sync_copy(idxf_hbm.at[pl.ds(tok0 * L, TPW * L)], idx_v)
    pltpu.sync_copy(acts_hbm.at[pl.ds(tok0, TPW)], a_v)

    pltpu.async_copy(wout_hbm.at[idx_v.at[pl.ds(0, L)]], g_v0, semg0)

    def grp_body(q, _):
        for j in range(4):
            t = q * 4 + j
            par = j % 2
            g_v = gbufs[par]
            tn = jnp.minimum(t + 1, TPW - 1)
            idx_n = idx_v.at[pl.ds(tn * L, L)]
            pltpu.async_copy(wout_hbm.at[idx_n], gbufs[1 - par],
                             gsems[1 - par])
            pltpu.make_async_copy(wout_hbm.at[idx_n], g_v,
                                  gsems[par]).wait()
            av = a_v[t, :]

            def kb(k, _, j=j, av=av, g_v=g_v):
                for u in range(4):
                    s = pl.ds((k * 4 + u) * L, L)
                    acc = av[0] * g_v[0, s]
                    for l in range(1, N_LEVELS):
                        acc = acc + av[l] * g_v[l, s]
                    out_v[j, s] = acc
                return 0
            lax.fori_loop(0, KCH // 4, kb, 0)

        pltpu.sync_copy(out_v, out_hbm.at[pl.ds(tok0 + q * 4, 4)])
        return 0

    lax.fori_loop(0, TPW // 4, grp_body, 0)
    pltpu.make_async_copy(wout_hbm.at[idx_v.at[pl.ds(0, L)]], g_v0,
                          semg0).wait()


def _mesh():
    return plsc.VectorSubcoreMesh(core_axis_name="c", subcore_axis_name="s")


@jax.jit
def _fff_sc(xf, in_weight, in_bias, out_weight):
    level = pl.kernel(
        _level_body,
        out_type=(jax.ShapeDtypeStruct((T_TOTAL,), jnp.float32),
                  jax.ShapeDtypeStruct((T_TOTAL,), jnp.float32)),
        mesh=_mesh(),
        scratch_types=[
            pltpu.VMEM((HALF, WIDTH), jnp.float32),
            pltpu.VMEM((HALF, WIDTH), jnp.float32),
            pltpu.VMEM((HALF, WIDTH), jnp.float32),
            pltpu.VMEM((HALF, WIDTH), jnp.float32),
            pltpu.VMEM((TPW,), jnp.int32),
            pltpu.VMEM((TPW,), jnp.float32),
            pltpu.VMEM((TPW,), jnp.float32),
            pltpu.VMEM((TPW,), jnp.float32),
            pltpu.SemaphoreType.DMA,
            pltpu.SemaphoreType.DMA,
            pltpu.SemaphoreType.DMA,
            pltpu.SemaphoreType.DMA,
        ],
    )

    idx = jnp.zeros((T_TOTAL,), jnp.int32)
    acts = []
    idxs = []
    for _ in range(N_LEVELS):
        bvals = in_bias[idx]
        act, lg = level(xf, in_weight, bvals, idx)
        acts.append(act)
        idxs.append(idx)
        idx = 2 * idx + 1 + (lg > 0).astype(jnp.int32)

    idx_pad = jnp.zeros((T_TOTAL, L), jnp.int32)
    idx_pad = idx_pad.at[:, :N_LEVELS].set(jnp.stack(idxs, axis=1))
    idx_flat = idx_pad.reshape(-1)
    acts_pad = jnp.zeros((T_TOTAL, L), jnp.float32)
    acts_pad = acts_pad.at[:, :N_LEVELS].set(jnp.stack(acts, axis=1))

    out = pl.kernel(
        _out_body,
        out_type=jax.ShapeDtypeStruct((T_TOTAL, OUT_WIDTH), jnp.float32),
        mesh=_mesh(),
        scratch_types=[
            pltpu.VMEM((L, OUT_WIDTH), jnp.float32),
            pltpu.VMEM((L, OUT_WIDTH), jnp.float32),
            pltpu.VMEM((4, OUT_WIDTH), jnp.float32),
            pltpu.VMEM((TPW * L,), jnp.int32),
            pltpu.VMEM((TPW, L), jnp.float32),
            pltpu.SemaphoreType.DMA,
            pltpu.SemaphoreType.DMA,
        ],
    )(out_weight, idx_flat, acts_pad)
    return out


def kernel(input, in_weight, in_bias, out_weight):
    xf = input.reshape(-1, WIDTH)
    out = _fff_sc(xf, in_weight, in_bias, out_weight)
    return out.reshape(input.shape[0], input.shape[1], OUT_WIDTH)


# speedup vs baseline: 1.9485x; 1.9485x over previous
import math

import jax
import jax.numpy as jnp
from jax import lax
from jax.experimental import pallas as pl
from jax.experimental.pallas import tpu as pltpu
from jax.experimental.pallas import tpu_sc as plsc

DEPTH = 11
N_NODES = 2 ** (DEPTH + 1) - 1
WIDTH = 2048
OUT_WIDTH = 2048
N_LEVELS = DEPTH + 1
TC_LEVELS = 7
SC_LEVELS = N_LEVELS - TC_LEVELS
NDENSE = 2 ** TC_LEVELS - 1
GPT = 8

L = 16
NW = 32
CHUNK = 16
HALF = 8
KCH = WIDTH // L
T_TOTAL = 8192
TPW = T_TOTAL // NW
UNROLL = 8
BT = 256
OCHUNK = 4


def _gelu_exact(z):
    u = z * jnp.float32(1.0 / math.sqrt(2.0))
    au = jnp.abs(u)
    t = jnp.float32(1.0) / (jnp.float32(1.0) + jnp.float32(0.3275911) * au)
    poly = jnp.float32(1.061405429)
    poly = poly * t + jnp.float32(-1.453152027)
    poly = poly * t + jnp.float32(1.421413741)
    poly = poly * t + jnp.float32(-0.284496736)
    poly = poly * t + jnp.float32(0.254829592)
    poly = poly * t
    erf_abs = jnp.float32(1.0) - poly * jnp.exp(-(au * au))
    erf = jnp.where(u >= 0, erf_abs, -erf_abs)
    return z * jnp.float32(0.5) * (jnp.float32(1.0) + erf)


def _tc_body(x_ref, win_ref, b_ref, wout_ref, out_ref, idx_ref):
    X = x_ref[...]
    LG = lax.dot_general(X, win_ref[...], (((1,), (1,)), ((), ())),
                         precision=lax.Precision.HIGHEST,
                         preferred_element_type=jnp.float32)
    LG = LG + b_ref[...]
    cols = lax.broadcasted_iota(jnp.int32, (BT, 128), 1)
    cur = jnp.zeros((BT, 1), jnp.int32)
    S = jnp.zeros((BT, 128), jnp.float32)
    for _ in range(TC_LEVELS):
        oh = cols == cur
        logit = jnp.sum(jnp.where(oh, LG, 0.0), axis=1, keepdims=True)
        act = _gelu_exact(logit)
        S = S + jnp.where(oh, act, 0.0)
        cur = 2 * cur + 1 + (logit > 0).astype(jnp.int32)
    out_ref[...] = lax.dot_general(S, wout_ref[...], (((1,), (0,)), ((), ())),
                                   precision=lax.Precision.HIGHEST,
                                   preferred_element_type=jnp.float32)
    idx_ref[0, 0, :] = cur[:, 0]


def _tc_front(xf, win0, b0, wout0):
    nblk = T_TOTAL // BT
    return pl.pallas_call(
        _tc_body,
        grid=(nblk,),
        in_specs=[
            pl.BlockSpec((BT, WIDTH), lambda i: (i, 0)),
            pl.BlockSpec((128, WIDTH), lambda i: (0, 0)),
            pl.BlockSpec((1, 128), lambda i: (0, 0)),
            pl.BlockSpec((128, OUT_WIDTH), lambda i: (0, 0)),
        ],
        out_specs=[
            pl.BlockSpec((BT, OUT_WIDTH), lambda i: (i, 0)),
            pl.BlockSpec((1, 1, BT), lambda i: (i, 0, 0)),
        ],
        out_shape=[
            jax.ShapeDtypeStruct((T_TOTAL, OUT_WIDTH), jnp.float32),
            jax.ShapeDtypeStruct((nblk, 1, BT), jnp.int32),
        ],
    )(xf, win0, b0, wout0)


_DNUMS = lax.GatherDimensionNumbers(
    offset_dims=(), collapsed_slice_dims=(0,), start_index_map=(0,))


def _lane_sum_splat(v):
    lanes = jnp.arange(L, dtype=jnp.int32)
    for sh in (8, 4, 2, 1):
        idx = (lanes ^ sh)[:, None]
        v = v + lax.gather(v, idx, _DNUMS, (1,),
                           mode=lax.GatherScatterMode.PROMISE_IN_BOUNDS)
    return v


def _level_body(x_hbm, win_hbm, bvals_hbm, idx_hbm, act_hbm, lg_hbm,
                x_v0, x_v1, w_v0, w_v1, idx_v, bv_v, st_v, lst_v,
                semw0, semw1, semx0, semx1):
    info = plsc.get_sparse_core_info()
    wid = lax.axis_index("s") * info.num_cores + lax.axis_index("c")
    tok0 = wid * TPW
    lanes = jnp.arange(L, dtype=jnp.int32)
    NH = TPW // HALF
    wbufs = (w_v0, w_v1)
    xbufs = (x_v0, x_v1)
    wsems = (semw0, semw1)
    xsems = (semx0, semx1)

    pltpu.sync_copy(idx_hbm.at[pl.ds(tok0, TPW)], idx_v)
    pltpu.sync_copy(bvals_hbm.at[pl.ds(tok0, TPW)], bv_v)

    def issue(g, par):
        idx_g = idx_v.at[pl.ds(g * HALF, HALF)]
        pltpu.async_copy(win_hbm.at[idx_g], wbufs[par], wsems[par])
        pltpu.async_copy(x_hbm.at[pl.ds(tok0 + g * HALF, HALF)],
                         xbufs[par], xsems[par])

    issue(0, 0)

    def chunk_body(c, _):
        logits = jnp.zeros((L,), jnp.float32)
        for h in (0, 1):
            g = 2 * c + h
            w_v, x_v = wbufs[h], xbufs[h]
            gn = jnp.minimum(g + 1, NH - 1)
            issue(gn, 1 - h)
            idx_g = idx_v.at[pl.ds(gn * HALF, HALF)]
            pltpu.make_async_copy(win_hbm.at[idx_g], w_v, wsems[h]).wait()
            pltpu.make_async_copy(
                x_hbm.at[pl.ds(tok0, HALF)], x_v, xsems[h]).wait()

            for t in range(HALF):
                def kb(k, acc, t=t, w_v=w_v, x_v=x_v):
                    for u in range(UNROLL):
                        s = pl.ds((k * UNROLL + u) * L, L)
                        acc = acc + x_v[t, s] * w_v[t, s]
                    return acc
                acc = lax.fori_loop(0, KCH // UNROLL, kb,
                                    jnp.zeros((L,), jnp.float32))
                logits = jnp.where(lanes == h * HALF + t,
                                   _lane_sum_splat(acc), logits)

        lg = logits + bv_v[pl.ds(c * CHUNK, CHUNK)]
        act = _gelu_exact(lg)
        st_v[pl.ds(c * CHUNK, CHUNK)] = act
        lst_v[pl.ds(c * CHUNK, CHUNK)] = lg
        return 0

    lax.fori_loop(0, TPW // CHUNK, chunk_body, 0)
    idx0 = idx_v.at[pl.ds(0, HALF)]
    pltpu.make_async_copy(win_hbm.at[idx0], w_v0, semw0).wait()
    pltpu.make_async_copy(x_hbm.at[pl.ds(tok0, HALF)], x_v0, semx0).wait()
    pltpu.sync_copy(st_v, act_hbm.at[pl.ds(tok0, TPW)])
    pltpu.sync_copy(lst_v, lg_hbm.at[pl.ds(tok0, TPW)])


def _out_body(wout_hbm, idxf_hbm, acts_hbm, out0_hbm, out_hbm,
              g_v, out_v, idx_v, a_v, semg):
    info = plsc.get_sparse_core_info()
    wid = lax.axis_index("s") * info.num_cores + lax.axis_index("c")
    tok0 = wid * TPW
    R = OCHUNK * GPT

    pltpu.sync_copy(idxf_hbm.at[pl.ds(tok0 * GPT, TPW * GPT)], idx_v)
    pltpu.sync_copy(acts_hbm.at[pl.ds(tok0, TPW)], a_v)

    def chunk_body(c, _):
        base = tok0 + c * OCHUNK
        cp = pltpu.async_copy(
            wout_hbm.at[idx_v.at[pl.ds(c * R, R)]], g_v, semg)
        pltpu.sync_copy(out0_hbm.at[pl.ds(base, OCHUNK)], out_v)
        cp.wait()

        for t in range(OCHUNK):
            av = a_v[c * OCHUNK + t, :]

            def kb(k, _, t=t, av=av):
                for u in range(4):
                    s = pl.ds((k * 4 + u) * L, L)
                    acc = out_v[t, s] + av[0] * g_v[t * GPT + 0, s]
                    for l in range(1, SC_LEVELS):
                        acc = acc + av[l] * g_v[t * GPT + l, s]
                    out_v[t, s] = acc
                return 0
            lax.fori_loop(0, KCH // 4, kb, 0)

        pltpu.sync_copy(out_v, out_hbm.at[pl.ds(base, OCHUNK)])
        return 0

    lax.fori_loop(0, TPW // OCHUNK, chunk_body, 0)


def _mesh():
    return plsc.VectorSubcoreMesh(core_axis_name="c", subcore_axis_name="s")


@jax.jit
def _fff(xf, in_weight, in_bias, out_weight, win0, b0, wout0):
    out0, idx7_3d = _tc_front(xf, win0, b0, wout0)
    idx = idx7_3d.reshape(T_TOTAL)

    level = pl.kernel(
        _level_body,
        out_type=(jax.ShapeDtypeStruct((T_TOTAL,), jnp.float32),
                  jax.ShapeDtypeStruct((T_TOTAL,), jnp.float32)),
        mesh=_mesh(),
        scratch_types=[
            pltpu.VMEM((HALF, WIDTH), jnp.float32),
            pltpu.VMEM((HALF, WIDTH), jnp.float32),
            pltpu.VMEM((HALF, WIDTH), jnp.float32),
            pltpu.VMEM((HALF, WIDTH), jnp.float32),
            pltpu.VMEM((TPW,), jnp.int32),
            pltpu.VMEM((TPW,), jnp.float32),
            pltpu.VMEM((TPW,), jnp.float32),
            pltpu.VMEM((TPW,), jnp.float32),
            pltpu.SemaphoreType.DMA,
            pltpu.SemaphoreType.DMA,
            pltpu.SemaphoreType.DMA,
            pltpu.SemaphoreType.DMA,
        ],
    )

    acts = []
    idxs = []
    for _ in range(SC_LEVELS):
        bvals = in_bias[idx]
        act, lg = level(xf, in_weight, bvals, idx)
        acts.append(act)
        idxs.append(idx)
        idx = 2 * idx + 1 + (lg > 0).astype(jnp.int32)

    idx_pad = jnp.zeros((T_TOTAL, GPT), jnp.int32)
    idx_pad = idx_pad.at[:, :SC_LEVELS].set(jnp.stack(idxs, axis=1))
    idx_flat = idx_pad.reshape(-1)
    acts_pad = jnp.zeros((T_TOTAL, L), jnp.float32)
    acts_pad = acts_pad.at[:, :SC_LEVELS].set(jnp.stack(acts, axis=1))

    out = pl.kernel(
        _out_body,
        out_type=jax.ShapeDtypeStruct((T_TOTAL, OUT_WIDTH), jnp.float32),
        mesh=_mesh(),
        scratch_types=[
            pltpu.VMEM((OCHUNK * GPT, OUT_WIDTH), jnp.float32),
            pltpu.VMEM((OCHUNK, OUT_WIDTH), jnp.float32),
            pltpu.VMEM((TPW * GPT,), jnp.int32),
            pltpu.VMEM((TPW, L), jnp.float32),
            pltpu.SemaphoreType.DMA,
        ],
    )(out_weight, idx_flat, acts_pad, out0)
    return out


def kernel(input, in_weight, in_bias, out_weight):
    xf = input.reshape(-1, WIDTH)
    win0 = jnp.zeros((128, WIDTH), jnp.float32)
    win0 = win0.at[:NDENSE].set(in_weight[:NDENSE])
    b0 = jnp.zeros((1, 128), jnp.float32)
    b0 = b0.at[0, :NDENSE].set(in_bias[:NDENSE])
    wout0 = jnp.zeros((128, OUT_WIDTH), jnp.float32)
    wout0 = wout0.at[:NDENSE].set(out_weight[:NDENSE])
    out = _fff(xf, in_weight, in_bias, out_weight, win0, b0, wout0)
    return out.reshape(input.shape[0], input.shape[1], OUT_WIDTH)


# speedup vs baseline: 4.0900x; 2.0991x over previous
import math

import jax
import jax.numpy as jnp
from jax import lax
from jax.experimental import pallas as pl
from jax.experimental.pallas import tpu as pltpu
from jax.experimental.pallas import tpu_sc as plsc

DEPTH = 11
N_NODES = 2 ** (DEPTH + 1) - 1
WIDTH = 2048
OUT_WIDTH = 2048
N_LEVELS = DEPTH + 1
TC_LEVELS = 7
SC_LEVELS = N_LEVELS - TC_LEVELS
NDENSE = 2 ** TC_LEVELS - 1
GPT = 8

L = 16
NW = 32
CHUNK = 16
HALF = 8
KCH = WIDTH // L
T_TOTAL = 8192
TPW = T_TOTAL // NW
UNROLL = 8
BT = 256
OCHUNK = 4


def _gelu_exact(z):
    u = z * jnp.float32(1.0 / math.sqrt(2.0))
    au = jnp.abs(u)
    t = jnp.float32(1.0) / (jnp.float32(1.0) + jnp.float32(0.3275911) * au)
    poly = jnp.float32(1.061405429)
    poly = poly * t + jnp.float32(-1.453152027)
    poly = poly * t + jnp.float32(1.421413741)
    poly = poly * t + jnp.float32(-0.284496736)
    poly = poly * t + jnp.float32(0.254829592)
    poly = poly * t
    erf_abs = jnp.float32(1.0) - poly * jnp.exp(-(au * au))
    erf = jnp.where(u >= 0, erf_abs, -erf_abs)
    return z * jnp.float32(0.5) * (jnp.float32(1.0) + erf)


def _tc_body(x_ref, win_ref, b_ref, wout_ref, out_ref, idx_ref):
    X = x_ref[...]
    LG = lax.dot_general(X, win_ref[...], (((1,), (1,)), ((), ())),
                         precision=lax.Precision.HIGHEST,
                         preferred_element_type=jnp.float32)
    LG = LG + b_ref[...]
    cols = lax.broadcasted_iota(jnp.int32, (BT, 128), 1)
    cur = jnp.zeros((BT, 1), jnp.int32)
    S = jnp.zeros((BT, 128), jnp.float32)
    for _ in range(TC_LEVELS):
        oh = cols == cur
        logit = jnp.sum(jnp.where(oh, LG, 0.0), axis=1, keepdims=True)
        act = _gelu_exact(logit)
        S = S + jnp.where(oh, act, 0.0)
        cur = 2 * cur + 1 + (logit > 0).astype(jnp.int32)
    out_ref[...] = lax.dot_general(S, wout_ref[...], (((1,), (0,)), ((), ())),
                                   preferred_element_type=jnp.float32)
    idx_ref[0, 0, :] = cur[:, 0]


def _tc_front(xf, win0, b0, wout0):
    nblk = T_TOTAL // BT
    return pl.pallas_call(
        _tc_body,
        grid=(nblk,),
        in_specs=[
            pl.BlockSpec((BT, WIDTH), lambda i: (i, 0)),
            pl.BlockSpec((128, WIDTH), lambda i: (0, 0)),
            pl.BlockSpec((1, 128), lambda i: (0, 0)),
            pl.BlockSpec((128, OUT_WIDTH), lambda i: (0, 0)),
        ],
        out_specs=[
            pl.BlockSpec((BT, OUT_WIDTH), lambda i: (i, 0)),
            pl.BlockSpec((1, 1, BT), lambda i: (i, 0, 0)),
        ],
        out_shape=[
            jax.ShapeDtypeStruct((T_TOTAL, OUT_WIDTH), jnp.float32),
            jax.ShapeDtypeStruct((nblk, 1, BT), jnp.int32),
        ],
    )(xf, win0, b0, wout0)


_DNUMS = lax.GatherDimensionNumbers(
    offset_dims=(), collapsed_slice_dims=(0,), start_index_map=(0,))


def _lane_sum_splat(v):
    lanes = jnp.arange(L, dtype=jnp.int32)
    for sh in (8, 4, 2, 1):
        idx = (lanes ^ sh)[:, None]
        v = v + lax.gather(v, idx, _DNUMS, (1,),
                           mode=lax.GatherScatterMode.PROMISE_IN_BOUNDS)
    return v


def _level_body(x_hbm, win_hbm, bvals_hbm, idx_hbm, act_hbm, lg_hbm,
                x_v0, x_v1, w_v0, w_v1, idx_v, bv_v, st_v, lst_v,
                semw0, semw1, semx0, semx1):
    info = plsc.get_sparse_core_info()
    wid = lax.axis_index("s") * info.num_cores + lax.axis_index("c")
    tok0 = wid * TPW
    lanes = jnp.arange(L, dtype=jnp.int32)
    NH = TPW // HALF
    wbufs = (w_v0, w_v1)
    xbufs = (x_v0, x_v1)
    wsems = (semw0, semw1)
    xsems = (semx0, semx1)

    pltpu.sync_copy(idx_hbm.at[pl.ds(tok0, TPW)], idx_v)
    pltpu.sync_copy(bvals_hbm.at[pl.ds(tok0, TPW)], bv_v)

    def issue(g, par):
        idx_g = idx_v.at[pl.ds(g * HALF, HALF)]
        pltpu.async_copy(win_hbm.at[idx_g], wbufs[par], wsems[par])
        pltpu.async_copy(x_hbm.at[pl.ds(tok0 + g * HALF, HALF)],
                         xbufs[par], xsems[par])

    issue(0, 0)

    def chunk_body(c, _):
        logits = jnp.zeros((L,), jnp.float32)
        for h in (0, 1):
            g = 2 * c + h
            w_v, x_v = wbufs[h], xbufs[h]
            gn = jnp.minimum(g + 1, NH - 1)
            issue(gn, 1 - h)
            idx_g = idx_v.at[pl.ds(gn * HALF, HALF)]
            pltpu.make_async_copy(win_hbm.at[idx_g], w_v, wsems[h]).wait()
            pltpu.make_async_copy(
                x_hbm.at[pl.ds(tok0, HALF)], x_v, xsems[h]).wait()

            for t in range(HALF):
                def kb(k, acc, t=t, w_v=w_v, x_v=x_v):
                    for u in range(UNROLL):
                        s = pl.ds((k * UNROLL + u) * L, L)
                        acc = acc + x_v[t, s] * w_v[t, s]
                    return acc
                acc = lax.fori_loop(0, KCH // UNROLL, kb,
                                    jnp.zeros((L,), jnp.float32))
                logits = jnp.where(lanes == h * HALF + t,
                                   _lane_sum_splat(acc), logits)

        lg = logits + bv_v[pl.ds(c * CHUNK, CHUNK)]
        act = _gelu_exact(lg)
        st_v[pl.ds(c * CHUNK, CHUNK)] = act
        lst_v[pl.ds(c * CHUNK, CHUNK)] = lg
        return 0

    lax.fori_loop(0, TPW // CHUNK, chunk_body, 0)
    idx0 = idx_v.at[pl.ds(0, HALF)]
    pltpu.make_async_copy(win_hbm.at[idx0], w_v0, semw0).wait()
    pltpu.make_async_copy(x_hbm.at[pl.ds(tok0, HALF)], x_v0, semx0).wait()
    pltpu.sync_copy(st_v, act_hbm.at[pl.ds(tok0, TPW)])
    pltpu.sync_copy(lst_v, lg_hbm.at[pl.ds(tok0, TPW)])


def _out_body(wout_hbm, idxf_hbm, acts_hbm, out0_hbm, out_hbm,
              g_v, out_v, idx_v, a_v, semg):
    info = plsc.get_sparse_core_info()
    wid = lax.axis_index("s") * info.num_cores + lax.axis_index("c")
    tok0 = wid * TPW
    R = 8 * SC_LEVELS

    pltpu.sync_copy(idxf_hbm.at[pl.ds(tok0 * SC_LEVELS, TPW * SC_LEVELS)],
                    idx_v)
    pltpu.sync_copy(acts_hbm.at[pl.ds(tok0, TPW)], a_v)

    def grp_body(c, _):
        base = tok0 + c * 8
        cp = pltpu.async_copy(
            wout_hbm.at[idx_v.at[pl.ds(c * R, R)]], g_v, semg)
        cp.wait()

        for half in (0, 1):
            pltpu.sync_copy(out0_hbm.at[pl.ds(base + half * 4, 4)], out_v)
            for t in range(4):
                tt = half * 4 + t
                av = a_v[c * 8 + tt, :]

                def kb(k, _, t=t, tt=tt, av=av):
                    for u in range(4):
                        s = pl.ds((k * 4 + u) * L, L)
                        acc = out_v[t, s] + av[0] * g_v[tt * SC_LEVELS + 0, s]
                        for l in range(1, SC_LEVELS):
                            acc = acc + av[l] * g_v[tt * SC_LEVELS + l, s]
                        out_v[t, s] = acc
                    return 0
                lax.fori_loop(0, KCH // 4, kb, 0)

            pltpu.sync_copy(out_v, out_hbm.at[pl.ds(base + half * 4, 4)])
        return 0

    lax.fori_loop(0, TPW // 8, grp_body, 0)


def _mesh():
    return plsc.VectorSubcoreMesh(core_axis_name="c", subcore_axis_name="s")


@jax.jit
def _fff(xf, in_weight, in_bias, out_weight, win0, b0, wout0):
    out0, idx7_3d = _tc_front(xf, win0, b0, wout0)
    idx = idx7_3d.reshape(T_TOTAL)

    level = pl.kernel(
        _level_body,
        out_type=(jax.ShapeDtypeStruct((T_TOTAL,), jnp.float32),
                  jax.ShapeDtypeStruct((T_TOTAL,), jnp.float32)),
        mesh=_mesh(),
        scratch_types=[
            pltpu.VMEM((HALF, WIDTH), jnp.float32),
            pltpu.VMEM((HALF, WIDTH), jnp.float32),
            pltpu.VMEM((HALF, WIDTH), jnp.float32),
            pltpu.VMEM((HALF, WIDTH), jnp.float32),
            pltpu.VMEM((TPW,), jnp.int32),
            pltpu.VMEM((TPW,), jnp.float32),
            pltpu.VMEM((TPW,), jnp.float32),
            pltpu.VMEM((TPW,), jnp.float32),
            pltpu.SemaphoreType.DMA,
            pltpu.SemaphoreType.DMA,
            pltpu.SemaphoreType.DMA,
            pltpu.SemaphoreType.DMA,
        ],
    )

    acts = []
    idxs = []
    for _ in range(SC_LEVELS):
        bvals = in_bias[idx]
        act, lg = level(xf, in_weight, bvals, idx)
        acts.append(act)
        idxs.append(idx)
        idx = 2 * idx + 1 + (lg > 0).astype(jnp.int32)

    idx_flat = jnp.stack(idxs, axis=1).reshape(-1)
    acts_pad = jnp.zeros((T_TOTAL, L), jnp.float32)
    acts_pad = acts_pad.at[:, :SC_LEVELS].set(jnp.stack(acts, axis=1))

    out = pl.kernel(
        _out_body,
        out_type=jax.ShapeDtypeStruct((T_TOTAL, OUT_WIDTH), jnp.float32),
        mesh=_mesh(),
        scratch_types=[
            pltpu.VMEM((8 * SC_LEVELS, OUT_WIDTH), jnp.float32),
            pltpu.VMEM((4, OUT_WIDTH), jnp.float32),
            pltpu.VMEM((TPW * SC_LEVELS,), jnp.int32),
            pltpu.VMEM((TPW, L), jnp.float32),
            pltpu.SemaphoreType.DMA,
        ],
    )(out_weight, idx_flat, acts_pad, out0)
    return out


def kernel(input, in_weight, in_bias, out_weight):
    xf = input.reshape(-1, WIDTH)
    win0 = jnp.zeros((128, WIDTH), jnp.float32)
    win0 = win0.at[:NDENSE].set(in_weight[:NDENSE])
    b0 = jnp.zeros((1, 128), jnp.float32)
    b0 = b0.at[0, :NDENSE].set(in_bias[:NDENSE])
    wout0 = jnp.zeros((128, OUT_WIDTH), jnp.float32)
    wout0 = wout0.at[:NDENSE].set(out_weight[:NDENSE])
    out = _fff(xf, in_weight, in_bias, out_weight, win0, b0, wout0)
    return out.reshape(input.shape[0], input.shape[1], OUT_WIDTH)
